# baseline (speedup 1.0000x reference)
import functools

import jax
import jax.numpy as jnp
import numpy as np
from jax import lax
from jax.experimental import pallas as pl
from jax.experimental.pallas import tpu as pltpu
from jax.experimental.pallas import tpu_sc as plsc

N_NODES_C = 50000
N_GRAPHS_C = 200
HIDDEN_C = 16
MAX_ELEM_C = 500

_NC, _NS = 2, 16
_NW = _NC * _NS
_CH = 1024
_ACC_PER_TILE = 3136
_ACC_ROWS = _ACC_PER_TILE * _NS


def _sc_edge_aggregate(src2d, dst2d, table3, split_features):
    e_pad = src2d.shape[0] * 128
    ntiles = _NS if split_features else _NW
    chunks = e_pad // (ntiles * _CH)
    rows_per_tile = chunks * (_CH // 128)
    mesh = plsc.VectorSubcoreMesh(core_axis_name="c", subcore_axis_name="s")

    @functools.partial(
        pl.kernel,
        out_type=jax.ShapeDtypeStruct((_NC, _ACC_ROWS, 16), jnp.float32),
        mesh=mesh,
        scratch_types=[
            pltpu.VMEM((8, 128), jnp.int32),
            pltpu.VMEM((8, 128), jnp.int32),
            pltpu.VMEM((_CH, 16), jnp.float32),
            pltpu.VMEM((784, 16), jnp.float32),
            pltpu.VMEM_SHARED((_ACC_ROWS, 16), jnp.float32),
            pltpu.SemaphoreType.DMA,
        ],
        compiler_params=pltpu.CompilerParams(use_tc_tiling_on_sc=False),
    )
    def k(src_hbm, dst_hbm, table_hbm, out_hbm, sidx, didx, rows, zbuf, acc, sem):
        ci = lax.axis_index("c")
        si = lax.axis_index("s")
        tid = si if split_features else si * _NC + ci
        tbl = table_hbm.at[ci] if split_features else table_hbm.at[0]

        @pl.loop(0, 784)
        def _(i):
            zbuf[i, :] = jnp.zeros((16,), jnp.float32)

        for q in range(4):
            pltpu.sync_copy(zbuf, acc.at[pl.ds(si * _ACC_PER_TILE + q * 784, 784)])
        plsc.subcore_barrier()

        @pl.loop(0, chunks)
        def _(c):
            row_base = tid * rows_per_tile + c * 8
            pltpu.sync_copy(src_hbm.at[pl.ds(row_base, 8)], sidx)
            pltpu.sync_copy(dst_hbm.at[pl.ds(row_base, 8)], didx)
            cps = [
                pltpu.async_copy(
                    tbl.at[sidx.at[j]], rows.at[pl.ds(j * 128, 128)], sem)
                for j in range(8)
            ]
            for cp in cps:
                cp.wait()
            for j in range(8):
                pltpu.sync_copy(
                    rows.at[pl.ds(j * 128, 128)], acc.at[didx.at[j]], add=True)

        plsc.subcore_barrier()

        pltpu.sync_copy(
            acc.at[pl.ds(si * _ACC_PER_TILE, _ACC_PER_TILE)],
            out_hbm.at[ci].at[pl.ds(si * _ACC_PER_TILE, _ACC_PER_TILE)])

    return k(src2d, dst2d, table3)


def _pad_edges(src, dst):
    e = src.shape[0]
    unit = _NW * _CH
    e_pad = ((e + unit - 1) // unit) * unit
    src2d = jnp.pad(src, (0, e_pad - e)).reshape(-1, 128)
    dst2d = jnp.pad(dst, (0, e_pad - e),
                    constant_values=N_NODES_C).reshape(-1, 128)
    return src2d, dst2d


_HI = jax.lax.Precision.HIGHEST


def _dot(a, b, dims):
    return lax.dot_general(a, b, (dims, ((), ())),
                           preferred_element_type=jnp.float32, precision=_HI)


def _mm(a, b):
    return _dot(a, b, ((1,), (0,)))


def _counts_body(batch_ref, lt_ref, counts_ref, starts_ref, l_ref):
    nblk = batch_ref.shape[0] // 1024

    def body(b, acc):
        vals = batch_ref[pl.ds(b * 1024, 1024), :]
        oh = (vals == lax.broadcasted_iota(jnp.int32, (1024, N_GRAPHS_C), 1))
        return acc + jnp.sum(oh.astype(jnp.float32), axis=0, keepdims=True)

    counts_f = lax.fori_loop(0, nblk, body, jnp.zeros((1, N_GRAPHS_C), jnp.float32))
    starts_f = _mm(counts_f, lt_ref[...])
    counts_ref[...] = counts_f.astype(jnp.int32)
    starts_ref[...] = starts_f.astype(jnp.int32)
    l_ref[...] = jnp.max(counts_f).astype(jnp.int32).reshape(1, 1)


def _graph_counts(batch):
    n = batch.shape[0]
    npad = ((n + 1023) // 1024) * 1024
    batch2d = jnp.pad(batch, (0, npad - n), constant_values=N_GRAPHS_C + 7)
    batch2d = batch2d.reshape(-1, 1)
    lt = jnp.asarray(np.triu(np.ones((N_GRAPHS_C, N_GRAPHS_C), np.float32), 1))
    return pl.pallas_call(
        _counts_body,
        out_shape=[
            jax.ShapeDtypeStruct((1, N_GRAPHS_C), jnp.int32),
            jax.ShapeDtypeStruct((1, N_GRAPHS_C), jnp.int32),
            jax.ShapeDtypeStruct((1, 1), jnp.int32),
        ],
    )(batch2d, lt)


_NBLK = 1024


def _sage1_body(parts_ref, oh_ref, tnum_ref, tden_ref, wl_ref, bl_ref,
                vw_ref, t_ref, h1_ref, q_ref):
    hist = parts_ref[0] + parts_ref[1]
    num = _mm(hist, tnum_ref[...])
    den = _mm(hist, tden_ref[...])
    aggr = num / (den + 1e-16)
    h1 = jax.nn.relu(_mm(aggr, wl_ref[...]) + bl_ref[...]
                     + _mm(oh_ref[...], vw_ref[...]))
    e2 = jnp.exp(h1 * t_ref[0, 0])
    h1_ref[...] = h1
    q_ref[0] = e2 * h1
    q_ref[1] = e2


def _sage1(parts, onehot, tnum, tden, wl, bl, vw, t):
    nb = _ACC_ROWS // _NBLK
    return pl.pallas_call(
        _sage1_body,
        grid=(nb,),
        in_specs=[
            pl.BlockSpec((2, _NBLK, 16), lambda i: (0, i, 0)),
            pl.BlockSpec((_NBLK, 16), lambda i: (i, 0)),
            pl.BlockSpec((16, 16), lambda i: (0, 0)),
            pl.BlockSpec((16, 16), lambda i: (0, 0)),
            pl.BlockSpec((16, 16), lambda i: (0, 0)),
            pl.BlockSpec((1, 16), lambda i: (0, 0)),
            pl.BlockSpec((16, 16), lambda i: (0, 0)),
            pl.BlockSpec((1, 1), lambda i: (0, 0), memory_space=pltpu.SMEM),
        ],
        out_specs=[
            pl.BlockSpec((_NBLK, 16), lambda i: (i, 0)),
            pl.BlockSpec((2, _NBLK, 16), lambda i: (0, i, 0)),
        ],
        out_shape=[
            jax.ShapeDtypeStruct((_ACC_ROWS, 16), jnp.float32),
            jax.ShapeDtypeStruct((2, _ACC_ROWS, 16), jnp.float32),
        ],
    )(parts, onehot, tnum, tden, wl, bl, vw, t)


def _sage2_body(parts_ref, h1_ref, wl_ref, bl_ref, wr_ref, h2_ref):
    aggr = parts_ref[0] / (parts_ref[1] + 1e-16)
    h2_ref[...] = jax.nn.relu(_mm(aggr, wl_ref[...]) + bl_ref[...]
                              + _mm(h1_ref[...], wr_ref[...]))


def _sage2(parts, h1, wl, bl, wr):
    nb = _ACC_ROWS // _NBLK
    return pl.pallas_call(
        _sage2_body,
        grid=(nb,),
        in_specs=[
            pl.BlockSpec((2, _NBLK, 16), lambda i: (0, i, 0)),
            pl.BlockSpec((_NBLK, 16), lambda i: (i, 0)),
            pl.BlockSpec((16, 16), lambda i: (0, 0)),
            pl.BlockSpec((1, 16), lambda i: (0, 0)),
            pl.BlockSpec((16, 16), lambda i: (0, 0)),
        ],
        out_specs=pl.BlockSpec((_NBLK, 16), lambda i: (i, 0)),
        out_shape=jax.ShapeDtypeStruct((_ACC_ROWS, 16), jnp.float32),
    )(parts, h1, wl, bl, wr)


def _gru_body(dense_ref, wis_ref, whs_ref, bi_ref, bh_ref, l_ref, out_ref):
    H = HIDDEN_C
    G = N_GRAPHS_C
    bi = bi_ref[...]
    bh = bh_ref[...]
    wis = [wis_ref[k] for k in range(H)]
    whs = [whs_ref[k] for k in range(H)]

    def step(tt, hT):
        xtT = dense_ref[tt]
        giT = jnp.zeros((3 * H, G), jnp.float32) + bi
        ghT = jnp.zeros((3 * H, G), jnp.float32) + bh
        for k in range(H):
            giT = giT + wis[k] * xtT[k:k + 1, :]
            ghT = ghT + whs[k] * hT[k:k + 1, :]
        r = jax.nn.sigmoid(giT[0:H] + ghT[0:H])
        z = jax.nn.sigmoid(giT[H:2 * H] + ghT[H:2 * H])
        n = jnp.tanh(giT[2 * H:3 * H] + r * ghT[2 * H:3 * H])
        return (1.0 - z) * n + z * hT

    h0 = jnp.zeros((H, G), jnp.float32)
    out_ref[...] = lax.fori_loop(0, l_ref[0, 0], step, h0)


def _gru(dense_tT, wi, wh, bi, bh, l_arr):
    return pl.pallas_call(
        _gru_body,
        in_specs=[
            pl.BlockSpec(dense_tT.shape, lambda: (0, 0, 0)),
            pl.BlockSpec((16, 48, 1), lambda: (0, 0, 0)),
            pl.BlockSpec((16, 48, 1), lambda: (0, 0, 0)),
            pl.BlockSpec((48, 1), lambda: (0, 0)),
            pl.BlockSpec((48, 1), lambda: (0, 0)),
            pl.BlockSpec((1, 1), lambda: (0, 0), memory_space=pltpu.SMEM),
        ],
        out_specs=pl.BlockSpec((HIDDEN_C, N_GRAPHS_C), lambda: (0, 0)),
        out_shape=jax.ShapeDtypeStruct((HIDDEN_C, N_GRAPHS_C), jnp.float32),
    )(dense_tT, wi[:, :, None], wh[:, :, None], bi[:, None], bh[:, None], l_arr)


_GBLK = 8


def _attn_body(dense_ref, counts_ref, wq, bq, wk, bk, wv, bv, wo, bo,
               plw, plb, s_, pwq, pbq, pwk, pbk, pwv, pbv, pwo, pbo, out_ref):
    i = pl.program_id(0)
    ME = MAX_ELEM_C
    sq = _mm(s_[...], pwq[...]) + pbq[...]
    for g in range(_GBLK):
        c = counts_ref[i * _GBLK + g]
        rowi = lax.broadcasted_iota(jnp.int32, (ME, 16), 0)
        x = jnp.where(rowi < c, dense_ref[g], 0.0)
        qp = _mm(x, wq[...]) + bq[...]
        kp = _mm(x, wk[...]) + bk[...]
        vp = _mm(x, wv[...]) + bv[...]
        scores = _dot(qp, kp, ((1,), (1,))) * 0.25
        coli = lax.broadcasted_iota(jnp.int32, (ME, ME), 1)
        scores = jnp.where(coli < c, scores, -1e30)
        m = jnp.max(scores, axis=1, keepdims=True)
        e = jnp.exp(scores - m)
        a = e / jnp.sum(e, axis=1, keepdims=True)
        out = qp + _mm(a, vp)
        z2 = out + jax.nn.relu(_mm(out, wo[...]) + bo[...])
        kv = jax.nn.relu(_mm(z2, plw[...]) + plb[...])
        kp2 = _mm(kv, pwk[...]) + pbk[...]
        vp2 = _mm(kv, pwv[...]) + pbv[...]
        s2 = _dot(sq, kp2, ((1,), (1,))) * 0.25
        coli2 = lax.broadcasted_iota(jnp.int32, (1, ME), 1)
        s2 = jnp.where(coli2 < c, s2, -1e30)
        m2 = jnp.max(s2, axis=1, keepdims=True)
        e2 = jnp.exp(s2 - m2)
        a2 = e2 / jnp.sum(e2, axis=1, keepdims=True)
        o2 = sq + _mm(a2, vp2)
        st = o2 + jax.nn.relu(_mm(o2, pwo[...]) + pbo[...])
        st = jnp.where(st != st, 0.0, jnp.clip(st, -3.402823e38, 3.402823e38))
        out_ref[pl.ds(g, 1), :] = st


def _attention(dense3, counts, p):
    nb = N_GRAPHS_C // _GBLK
    w16 = lambda: pl.BlockSpec((16, 16), lambda i: (0, 0))
    b16 = lambda: pl.BlockSpec((1, 16), lambda i: (0, 0))
    return pl.pallas_call(
        _attn_body,
        grid=(nb,),
        in_specs=[
            pl.BlockSpec((_GBLK, MAX_ELEM_C, 16), lambda i: (i, 0, 0)),
            pl.BlockSpec(memory_space=pltpu.SMEM),
            w16(), b16(), w16(), b16(), w16(), b16(), w16(), b16(),
            w16(), b16(), b16(), w16(), b16(), w16(), b16(), w16(), b16(),
            w16(), b16(),
        ],
        out_specs=pl.BlockSpec((_GBLK, 16), lambda i: (i, 0)),
        out_shape=jax.ShapeDtypeStruct((N_GRAPHS_C, 16), jnp.float32),
    )(dense3, counts,
      p['enc_Wq'], p['enc_bq'][None], p['enc_Wk'], p['enc_bk'][None],
      p['enc_Wv'], p['enc_bv'][None], p['enc_Wo'], p['enc_bo'][None],
      p['pma_lin_W'], p['pma_lin_b'][None],
      p['S'].reshape(1, 16), p['pma_Wq'], p['pma_bq'][None],
      p['pma_Wk'], p['pma_bk'][None], p['pma_Wv'], p['pma_bv'][None],
      p['pma_Wo'], p['pma_bo'][None])


def _mlp_final_body(df_ref, wmlp_ref, bmlp_ref, gru_ref, st_ref,
                    wf1_ref, wf2_ref, wf3_ref, bf_ref, out_ref):
    mlp = _mm(df_ref[...], wmlp_ref[...]) + bmlp_ref[...]
    out_ref[...] = (_mm(mlp, wf1_ref[...])
                    + _mm(gru_ref[...], wf2_ref[...])
                    + _mm(st_ref[...], wf3_ref[...])
                    + bf_ref[...])


def _mlp_final(dense_flat, wmlp, bmlp, gru, st, wfin, bfin):
    return pl.pallas_call(
        _mlp_final_body,
        out_shape=jax.ShapeDtypeStruct((N_GRAPHS_C, 48), jnp.float32),
    )(dense_flat, wmlp, bmlp[None], gru, st,
      wfin[0:16], wfin[16:32], wfin[32:48], bfin[None])


def kernel(params, x, edge_index, batch):
    p = params
    N = x.shape[0]
    G = N_GRAPHS_C
    ME = MAX_ELEM_C
    t = p['t']

    c0 = jnp.repeat(jnp.arange(3), 3)
    c1 = jnp.tile(jnp.arange(3), 3)
    V = jnp.concatenate([p['emb'][c0], c1[:, None].astype(jnp.float32)], axis=1)
    cls = x[:, 0] * 3 + x[:, 1]
    src, dst = edge_index[0], edge_index[1]
    src2d, dst2d = _pad_edges(src, dst)
    onehot = (cls[:, None] == jnp.arange(16)[None, :]).astype(jnp.float32)
    onehot = jnp.pad(onehot, ((0, _ACC_ROWS - N), (0, 0)))
    E1 = jnp.exp(V * t)
    tnum = jnp.zeros((16, 16), jnp.float32).at[:9, :4].set(E1 * V)
    tden = jnp.zeros((16, 16), jnp.float32).at[:9, :4].set(E1)
    wl1 = jnp.zeros((16, 16), jnp.float32).at[:4].set(p['Wl1'])
    vw = jnp.zeros((16, 16), jnp.float32).at[:9].set(V @ p['Wr1'])
    t11 = t.reshape(1, 1)

    counts, starts, l_arr = _graph_counts(batch)

    parts1 = _sc_edge_aggregate(src2d, dst2d, onehot[None], False)
    h1, q3 = _sage1(parts1, onehot, tnum, tden, wl1, p['bl1'][None], vw, t11)

    parts2 = _sc_edge_aggregate(src2d, dst2d, q3, True)
    h2 = _sage2(parts2, h1, p['Wl2'], p['bl2'][None], p['Wr2'])

    pidx = jnp.arange(ME)[None, :]
    gidx = starts[0][:, None] + pidx
    mask = pidx < counts[0][:, None]
    dense = jnp.where(mask[:, :, None],
                      h2[jnp.minimum(gidx, N - 1)], 0.0)

    gruT = _gru(jnp.transpose(dense, (1, 2, 0)), p['Wi'], p['Wh'], p['bi'],
                p['bh'], l_arr)
    st = _attention(dense, counts.reshape(G), p)
    return _mlp_final(dense.reshape(G, ME * HIDDEN_C), p['Wmlp'], p['bmlp'],
                      gruT.T, st, p['Wfin'], p['bfin'])


# speedup vs baseline: 1.1713x; 1.1713x over previous
import functools

import jax
import jax.numpy as jnp
import numpy as np
from jax import lax
from jax.experimental import pallas as pl
from jax.experimental.pallas import tpu as pltpu
from jax.experimental.pallas import tpu_sc as plsc

N_NODES_C = 50000
N_GRAPHS_C = 200
HIDDEN_C = 16
MAX_ELEM_C = 500

_NC, _NS = 2, 16
_NW = _NC * _NS
_CH = 1024
_ACC_PER_TILE = 3136
_ACC_ROWS = _ACC_PER_TILE * _NS


def _sc_edge_aggregate(src2d, dst2d, table3, split_features):
    e_pad = src2d.shape[0] * 128
    ntiles = _NS if split_features else _NW
    chunks = e_pad // (ntiles * _CH)
    rows_per_tile = chunks * (_CH // 128)
    mesh = plsc.VectorSubcoreMesh(core_axis_name="c", subcore_axis_name="s")

    @functools.partial(
        pl.kernel,
        out_type=jax.ShapeDtypeStruct((_NC, _ACC_ROWS, 16), jnp.float32),
        mesh=mesh,
        scratch_types=[
            pltpu.VMEM((8, 128), jnp.int32),
            pltpu.VMEM((8, 128), jnp.int32),
            pltpu.VMEM((_CH, 16), jnp.float32),
            pltpu.VMEM((784, 16), jnp.float32),
            pltpu.VMEM_SHARED((_ACC_ROWS, 16), jnp.float32),
            pltpu.SemaphoreType.DMA,
        ],
        compiler_params=pltpu.CompilerParams(use_tc_tiling_on_sc=False),
    )
    def k(src_hbm, dst_hbm, table_hbm, out_hbm, sidx, didx, rows, zbuf, acc, sem):
        ci = lax.axis_index("c")
        si = lax.axis_index("s")
        tid = si if split_features else si * _NC + ci
        tbl = table_hbm.at[ci] if split_features else table_hbm.at[0]

        @pl.loop(0, 784)
        def _(i):
            zbuf[i, :] = jnp.zeros((16,), jnp.float32)

        for q in range(4):
            pltpu.sync_copy(zbuf, acc.at[pl.ds(si * _ACC_PER_TILE + q * 784, 784)])
        plsc.subcore_barrier()

        @pl.loop(0, chunks)
        def _(c):
            row_base = tid * rows_per_tile + c * 8
            pltpu.sync_copy(src_hbm.at[pl.ds(row_base, 8)], sidx)
            pltpu.sync_copy(dst_hbm.at[pl.ds(row_base, 8)], didx)
            cps = [
                pltpu.async_copy(
                    tbl.at[sidx.at[j]], rows.at[pl.ds(j * 128, 128)], sem)
                for j in range(8)
            ]
            for cp in cps:
                cp.wait()
            for j in range(8):
                pltpu.sync_copy(
                    rows.at[pl.ds(j * 128, 128)], acc.at[didx.at[j]], add=True)

        plsc.subcore_barrier()

        pltpu.sync_copy(
            acc.at[pl.ds(si * _ACC_PER_TILE, _ACC_PER_TILE)],
            out_hbm.at[ci].at[pl.ds(si * _ACC_PER_TILE, _ACC_PER_TILE)])

    return k(src2d, dst2d, table3)


def _pad_edges(src, dst):
    e = src.shape[0]
    unit = _NW * _CH
    e_pad = ((e + unit - 1) // unit) * unit
    src2d = jnp.pad(src, (0, e_pad - e)).reshape(-1, 128)
    dst2d = jnp.pad(dst, (0, e_pad - e),
                    constant_values=N_NODES_C).reshape(-1, 128)
    return src2d, dst2d


_HI = jax.lax.Precision.HIGHEST


def _dot(a, b, dims):
    return lax.dot_general(a, b, (dims, ((), ())),
                           preferred_element_type=jnp.float32, precision=_HI)


def _mm(a, b):
    return _dot(a, b, ((1,), (0,)))


def _counts_body(batch_ref, lt_ref, counts_ref, starts_ref, l_ref):
    nblk = batch_ref.shape[0] // 1024

    def body(b, acc):
        vals = batch_ref[pl.ds(b * 1024, 1024), :]
        oh = (vals == lax.broadcasted_iota(jnp.int32, (1024, N_GRAPHS_C), 1))
        return acc + jnp.sum(oh.astype(jnp.float32), axis=0, keepdims=True)

    counts_f = lax.fori_loop(0, nblk, body, jnp.zeros((1, N_GRAPHS_C), jnp.float32))
    starts_f = _mm(counts_f, lt_ref[...])
    counts_ref[...] = counts_f.astype(jnp.int32)
    starts_ref[...] = starts_f.astype(jnp.int32)
    l_ref[...] = jnp.max(counts_f).astype(jnp.int32).reshape(1, 1)


def _graph_counts(batch):
    n = batch.shape[0]
    npad = ((n + 1023) // 1024) * 1024
    batch2d = jnp.pad(batch, (0, npad - n), constant_values=N_GRAPHS_C + 7)
    batch2d = batch2d.reshape(-1, 1)
    lt = jnp.asarray(np.triu(np.ones((N_GRAPHS_C, N_GRAPHS_C), np.float32), 1))
    return pl.pallas_call(
        _counts_body,
        out_shape=[
            jax.ShapeDtypeStruct((1, N_GRAPHS_C), jnp.int32),
            jax.ShapeDtypeStruct((1, N_GRAPHS_C), jnp.int32),
            jax.ShapeDtypeStruct((1, 1), jnp.int32),
        ],
    )(batch2d, lt)


_NBLK = 1024


def _sage1_body(parts_ref, oh_ref, tnum_ref, tden_ref, wl_ref, bl_ref,
                vw_ref, t_ref, h1_ref, q_ref):
    hist = parts_ref[0] + parts_ref[1]
    num = _mm(hist, tnum_ref[...])
    den = _mm(hist, tden_ref[...])
    aggr = num / (den + 1e-16)
    h1 = jax.nn.relu(_mm(aggr, wl_ref[...]) + bl_ref[...]
                     + _mm(oh_ref[...], vw_ref[...]))
    e2 = jnp.exp(h1 * t_ref[0, 0])
    h1_ref[...] = h1
    q_ref[0] = e2 * h1
    q_ref[1] = e2


def _sage1(parts, onehot, tnum, tden, wl, bl, vw, t):
    nb = _ACC_ROWS // _NBLK
    return pl.pallas_call(
        _sage1_body,
        grid=(nb,),
        in_specs=[
            pl.BlockSpec((2, _NBLK, 16), lambda i: (0, i, 0)),
            pl.BlockSpec((_NBLK, 16), lambda i: (i, 0)),
            pl.BlockSpec((16, 16), lambda i: (0, 0)),
            pl.BlockSpec((16, 16), lambda i: (0, 0)),
            pl.BlockSpec((16, 16), lambda i: (0, 0)),
            pl.BlockSpec((1, 16), lambda i: (0, 0)),
            pl.BlockSpec((16, 16), lambda i: (0, 0)),
            pl.BlockSpec((1, 1), lambda i: (0, 0), memory_space=pltpu.SMEM),
        ],
        out_specs=[
            pl.BlockSpec((_NBLK, 16), lambda i: (i, 0)),
            pl.BlockSpec((2, _NBLK, 16), lambda i: (0, i, 0)),
        ],
        out_shape=[
            jax.ShapeDtypeStruct((_ACC_ROWS, 16), jnp.float32),
            jax.ShapeDtypeStruct((2, _ACC_ROWS, 16), jnp.float32),
        ],
    )(parts, onehot, tnum, tden, wl, bl, vw, t)


def _sage2_body(parts_ref, h1_ref, wl_ref, bl_ref, wr_ref, h2_ref):
    aggr = parts_ref[0] / (parts_ref[1] + 1e-16)
    h2_ref[...] = jax.nn.relu(_mm(aggr, wl_ref[...]) + bl_ref[...]
                              + _mm(h1_ref[...], wr_ref[...]))


def _sage2(parts, h1, wl, bl, wr):
    nb = _ACC_ROWS // _NBLK
    return pl.pallas_call(
        _sage2_body,
        grid=(nb,),
        in_specs=[
            pl.BlockSpec((2, _NBLK, 16), lambda i: (0, i, 0)),
            pl.BlockSpec((_NBLK, 16), lambda i: (i, 0)),
            pl.BlockSpec((16, 16), lambda i: (0, 0)),
            pl.BlockSpec((1, 16), lambda i: (0, 0)),
            pl.BlockSpec((16, 16), lambda i: (0, 0)),
        ],
        out_specs=pl.BlockSpec((_NBLK, 16), lambda i: (i, 0)),
        out_shape=jax.ShapeDtypeStruct((_ACC_ROWS, 16), jnp.float32),
    )(parts, h1, wl, bl, wr)


def _gru_body(dense_ref, wis_ref, whs_ref, bi_ref, bh_ref, l_ref, out_ref):
    H = HIDDEN_C
    G = N_GRAPHS_C
    bi = bi_ref[...]
    bh = bh_ref[...]
    wis = [wis_ref[k] for k in range(H)]
    whs = [whs_ref[k] for k in range(H)]

    def step(tt, hT):
        xtT = dense_ref[tt]
        giT = jnp.zeros((3 * H, G), jnp.float32) + bi
        ghT = jnp.zeros((3 * H, G), jnp.float32) + bh
        for k in range(H):
            giT = giT + wis[k] * xtT[k:k + 1, :]
            ghT = ghT + whs[k] * hT[k:k + 1, :]
        r = jax.nn.sigmoid(giT[0:H] + ghT[0:H])
        z = jax.nn.sigmoid(giT[H:2 * H] + ghT[H:2 * H])
        n = jnp.tanh(giT[2 * H:3 * H] + r * ghT[2 * H:3 * H])
        return (1.0 - z) * n + z * hT

    h0 = jnp.zeros((H, G), jnp.float32)
    out_ref[...] = lax.fori_loop(0, l_ref[0, 0], step, h0)


def _gru(dense_tT, wi, wh, bi, bh, l_arr):
    return pl.pallas_call(
        _gru_body,
        in_specs=[
            pl.BlockSpec(dense_tT.shape, lambda: (0, 0, 0)),
            pl.BlockSpec((16, 48, 1), lambda: (0, 0, 0)),
            pl.BlockSpec((16, 48, 1), lambda: (0, 0, 0)),
            pl.BlockSpec((48, 1), lambda: (0, 0)),
            pl.BlockSpec((48, 1), lambda: (0, 0)),
            pl.BlockSpec((1, 1), lambda: (0, 0), memory_space=pltpu.SMEM),
        ],
        out_specs=pl.BlockSpec((HIDDEN_C, N_GRAPHS_C), lambda: (0, 0)),
        out_shape=jax.ShapeDtypeStruct((HIDDEN_C, N_GRAPHS_C), jnp.float32),
    )(dense_tT, wi[:, :, None], wh[:, :, None], bi[:, None], bh[:, None], l_arr)


_GBLK = 8


def _attn_body(dense_ref, counts_ref, wq, bq, wk, bk, wv, bv, wo, bo,
               plw, plb, s_, pwq, pbq, pwk, pbk, pwv, pbv, pwo, pbo, out_ref):
    i = pl.program_id(0)
    ME = MAX_ELEM_C
    sq = _mm(s_[...], pwq[...]) + pbq[...]
    for g in range(_GBLK):
        c = counts_ref[i * _GBLK + g]
        rowi = lax.broadcasted_iota(jnp.int32, (ME, 16), 0)
        x = jnp.where(rowi < c, dense_ref[g], 0.0)
        qp = _mm(x, wq[...]) + bq[...]
        kp = _mm(x, wk[...]) + bk[...]
        vp = _mm(x, wv[...]) + bv[...]
        scores = lax.dot_general(qp, kp, (((1,), (1,)), ((), ())),
                                 preferred_element_type=jnp.float32) * 0.25
        coli = lax.broadcasted_iota(jnp.int32, (ME, ME), 1)
        scores = jnp.where(coli < c, scores, -1e30)
        e = jnp.exp(scores)
        a = e / jnp.sum(e, axis=1, keepdims=True)
        out = qp + jnp.dot(a, vp, preferred_element_type=jnp.float32)
        z2 = out + jax.nn.relu(_mm(out, wo[...]) + bo[...])
        kv = jax.nn.relu(_mm(z2, plw[...]) + plb[...])
        kp2 = _mm(kv, pwk[...]) + pbk[...]
        vp2 = _mm(kv, pwv[...]) + pbv[...]
        s2 = _dot(sq, kp2, ((1,), (1,))) * 0.25
        coli2 = lax.broadcasted_iota(jnp.int32, (1, ME), 1)
        s2 = jnp.where(coli2 < c, s2, -1e30)
        m2 = jnp.max(s2, axis=1, keepdims=True)
        e2 = jnp.exp(s2 - m2)
        a2 = e2 / jnp.sum(e2, axis=1, keepdims=True)
        o2 = sq + _mm(a2, vp2)
        st = o2 + jax.nn.relu(_mm(o2, pwo[...]) + pbo[...])
        st = jnp.where(st != st, 0.0, jnp.clip(st, -3.402823e38, 3.402823e38))
        out_ref[pl.ds(g, 1), :] = st


def _attention(dense3, counts, p):
    nb = N_GRAPHS_C // _GBLK
    w16 = lambda: pl.BlockSpec((16, 16), lambda i: (0, 0))
    b16 = lambda: pl.BlockSpec((1, 16), lambda i: (0, 0))
    return pl.pallas_call(
        _attn_body,
        grid=(nb,),
        in_specs=[
            pl.BlockSpec((_GBLK, MAX_ELEM_C, 16), lambda i: (i, 0, 0)),
            pl.BlockSpec(memory_space=pltpu.SMEM),
            w16(), b16(), w16(), b16(), w16(), b16(), w16(), b16(),
            w16(), b16(), b16(), w16(), b16(), w16(), b16(), w16(), b16(),
            w16(), b16(),
        ],
        out_specs=pl.BlockSpec((_GBLK, 16), lambda i: (i, 0)),
        out_shape=jax.ShapeDtypeStruct((N_GRAPHS_C, 16), jnp.float32),
    )(dense3, counts,
      p['enc_Wq'], p['enc_bq'][None], p['enc_Wk'], p['enc_bk'][None],
      p['enc_Wv'], p['enc_bv'][None], p['enc_Wo'], p['enc_bo'][None],
      p['pma_lin_W'], p['pma_lin_b'][None],
      p['S'].reshape(1, 16), p['pma_Wq'], p['pma_bq'][None],
      p['pma_Wk'], p['pma_bk'][None], p['pma_Wv'], p['pma_bv'][None],
      p['pma_Wo'], p['pma_bo'][None])


def _mlp_final_body(df_ref, wmlp_ref, bmlp_ref, gru_ref, st_ref,
                    wf1_ref, wf2_ref, wf3_ref, bf_ref, out_ref):
    mlp = _mm(df_ref[...], wmlp_ref[...]) + bmlp_ref[...]
    out_ref[...] = (_mm(mlp, wf1_ref[...])
                    + _mm(gru_ref[...], wf2_ref[...])
                    + _mm(st_ref[...], wf3_ref[...])
                    + bf_ref[...])


def _mlp_final(dense_flat, wmlp, bmlp, gru, st, wfin, bfin):
    return pl.pallas_call(
        _mlp_final_body,
        out_shape=jax.ShapeDtypeStruct((N_GRAPHS_C, 48), jnp.float32),
    )(dense_flat, wmlp, bmlp[None], gru, st,
      wfin[0:16], wfin[16:32], wfin[32:48], bfin[None])


def kernel(params, x, edge_index, batch):
    p = params
    N = x.shape[0]
    G = N_GRAPHS_C
    ME = MAX_ELEM_C
    t = p['t']

    c0 = jnp.repeat(jnp.arange(3), 3)
    c1 = jnp.tile(jnp.arange(3), 3)
    V = jnp.concatenate([p['emb'][c0], c1[:, None].astype(jnp.float32)], axis=1)
    cls = x[:, 0] * 3 + x[:, 1]
    src, dst = edge_index[0], edge_index[1]
    src2d, dst2d = _pad_edges(src, dst)
    onehot = (cls[:, None] == jnp.arange(16)[None, :]).astype(jnp.float32)
    onehot = jnp.pad(onehot, ((0, _ACC_ROWS - N), (0, 0)))
    E1 = jnp.exp(V * t)
    tnum = jnp.zeros((16, 16), jnp.float32).at[:9, :4].set(E1 * V)
    tden = jnp.zeros((16, 16), jnp.float32).at[:9, :4].set(E1)
    wl1 = jnp.zeros((16, 16), jnp.float32).at[:4].set(p['Wl1'])
    vw = jnp.zeros((16, 16), jnp.float32).at[:9].set(V @ p['Wr1'])
    t11 = t.reshape(1, 1)

    counts, starts, l_arr = _graph_counts(batch)

    parts1 = _sc_edge_aggregate(src2d, dst2d, onehot[None], False)
    h1, q3 = _sage1(parts1, onehot, tnum, tden, wl1, p['bl1'][None], vw, t11)

    parts2 = _sc_edge_aggregate(src2d, dst2d, q3, True)
    h2 = _sage2(parts2, h1, p['Wl2'], p['bl2'][None], p['Wr2'])

    pidx = jnp.arange(ME)[None, :]
    gidx = starts[0][:, None] + pidx
    mask = pidx < counts[0][:, None]
    dense = jnp.where(mask[:, :, None],
                      h2[jnp.minimum(gidx, N - 1)], 0.0)

    gruT = _gru(jnp.transpose(dense, (1, 2, 0)), p['Wi'], p['Wh'], p['bi'],
                p['bh'], l_arr)
    st = _attention(dense, counts.reshape(G), p)
    return _mlp_final(dense.reshape(G, ME * HIDDEN_C), p['Wmlp'], p['bmlp'],
                      gruT.T, st, p['Wfin'], p['bfin'])


# speedup vs baseline: 1.3167x; 1.1242x over previous
import functools

import jax
import jax.numpy as jnp
import numpy as np
from jax import lax
from jax.experimental import pallas as pl
from jax.experimental.pallas import tpu as pltpu
from jax.experimental.pallas import tpu_sc as plsc

N_NODES_C = 50000
N_GRAPHS_C = 200
HIDDEN_C = 16
MAX_ELEM_C = 500

_NC, _NS = 2, 16
_NW = _NC * _NS
_CH = 1024
_ACC_PER_TILE = 3136
_ACC_ROWS = _ACC_PER_TILE * _NS


def _sc_edge_aggregate(src2d, dst2d, table3, split_features):
    e_pad = src2d.shape[0] * 128
    ntiles = _NS if split_features else _NW
    chunks = e_pad // (ntiles * _CH)
    rows_per_tile = chunks * (_CH // 128)
    mesh = plsc.VectorSubcoreMesh(core_axis_name="c", subcore_axis_name="s")

    @functools.partial(
        pl.kernel,
        out_type=jax.ShapeDtypeStruct((_NC, _ACC_ROWS, 16), jnp.float32),
        mesh=mesh,
        scratch_types=[
            pltpu.VMEM((16, 128), jnp.int32),
            pltpu.VMEM((16, 128), jnp.int32),
            pltpu.VMEM((_CH, 16), jnp.float32),
            pltpu.VMEM((_CH, 16), jnp.float32),
            pltpu.VMEM((784, 16), jnp.float32),
            pltpu.VMEM_SHARED((_ACC_ROWS, 16), jnp.float32),
            pltpu.SemaphoreType.DMA,
            pltpu.SemaphoreType.DMA,
        ],
        compiler_params=pltpu.CompilerParams(use_tc_tiling_on_sc=False),
    )
    def k(src_hbm, dst_hbm, table_hbm, out_hbm, sidx, didx, rows_a, rows_b,
          zbuf, acc, sem_a, sem_b):
        ci = lax.axis_index("c")
        si = lax.axis_index("s")
        tid = si if split_features else si * _NC + ci
        tbl = table_hbm.at[ci] if split_features else table_hbm.at[0]

        @pl.loop(0, 784)
        def _(i):
            zbuf[i, :] = jnp.zeros((16,), jnp.float32)

        for q in range(4):
            pltpu.sync_copy(zbuf, acc.at[pl.ds(si * _ACC_PER_TILE + q * 784, 784)])
        plsc.subcore_barrier()

        @pl.loop(0, chunks // 2)
        def _(cc):
            row_base = tid * rows_per_tile + cc * 16
            pltpu.sync_copy(src_hbm.at[pl.ds(row_base, 16)], sidx)
            pltpu.sync_copy(dst_hbm.at[pl.ds(row_base, 16)], didx)
            cps_a = [
                pltpu.async_copy(
                    tbl.at[sidx.at[j]], rows_a.at[pl.ds(j * 128, 128)], sem_a)
                for j in range(8)
            ]
            cps_b = [
                pltpu.async_copy(
                    tbl.at[sidx.at[8 + j]], rows_b.at[pl.ds(j * 128, 128)], sem_b)
                for j in range(8)
            ]
            for cp in cps_a:
                cp.wait()
            for j in range(8):
                pltpu.sync_copy(
                    rows_a.at[pl.ds(j * 128, 128)], acc.at[didx.at[j]], add=True)
            for cp in cps_b:
                cp.wait()
            for j in range(8):
                pltpu.sync_copy(
                    rows_b.at[pl.ds(j * 128, 128)], acc.at[didx.at[8 + j]],
                    add=True)

        plsc.subcore_barrier()

        pltpu.sync_copy(
            acc.at[pl.ds(si * _ACC_PER_TILE, _ACC_PER_TILE)],
            out_hbm.at[ci].at[pl.ds(si * _ACC_PER_TILE, _ACC_PER_TILE)])

    return k(src2d, dst2d, table3)


def _pad_edges(src, dst):
    e = src.shape[0]
    unit = _NW * _CH
    e_pad = ((e + unit - 1) // unit) * unit
    src2d = jnp.pad(src, (0, e_pad - e)).reshape(-1, 128)
    dst2d = jnp.pad(dst, (0, e_pad - e),
                    constant_values=N_NODES_C).reshape(-1, 128)
    return src2d, dst2d


_HI = jax.lax.Precision.HIGHEST


def _dot(a, b, dims):
    return lax.dot_general(a, b, (dims, ((), ())),
                           preferred_element_type=jnp.float32, precision=_HI)


def _mm(a, b):
    return _dot(a, b, ((1,), (0,)))


def _counts_body(batch_ref, lt_ref, counts_ref, starts_ref, l_ref):
    nblk = batch_ref.shape[0] // 1024

    def body(b, acc):
        vals = batch_ref[pl.ds(b * 1024, 1024), :]
        oh = (vals == lax.broadcasted_iota(jnp.int32, (1024, N_GRAPHS_C), 1))
        return acc + jnp.sum(oh.astype(jnp.float32), axis=0, keepdims=True)

    counts_f = lax.fori_loop(0, nblk, body, jnp.zeros((1, N_GRAPHS_C), jnp.float32))
    starts_f = _mm(counts_f, lt_ref[...])
    counts_ref[...] = counts_f.astype(jnp.int32)
    starts_ref[...] = starts_f.astype(jnp.int32)
    l_ref[...] = jnp.max(counts_f).astype(jnp.int32).reshape(1, 1)


def _graph_counts(batch):
    n = batch.shape[0]
    npad = ((n + 1023) // 1024) * 1024
    batch2d = jnp.pad(batch, (0, npad - n), constant_values=N_GRAPHS_C + 7)
    batch2d = batch2d.reshape(-1, 1)
    lt = jnp.asarray(np.triu(np.ones((N_GRAPHS_C, N_GRAPHS_C), np.float32), 1))
    return pl.pallas_call(
        _counts_body,
        out_shape=[
            jax.ShapeDtypeStruct((1, N_GRAPHS_C), jnp.int32),
            jax.ShapeDtypeStruct((1, N_GRAPHS_C), jnp.int32),
            jax.ShapeDtypeStruct((1, 1), jnp.int32),
        ],
    )(batch2d, lt)


_NBLK = 1024


def _sage1_body(parts_ref, oh_ref, tnum_ref, tden_ref, wl_ref, bl_ref,
                vw_ref, t_ref, h1_ref, q_ref):
    hist = parts_ref[0] + parts_ref[1]
    num = _mm(hist, tnum_ref[...])
    den = _mm(hist, tden_ref[...])
    aggr = num / (den + 1e-16)
    h1 = jax.nn.relu(_mm(aggr, wl_ref[...]) + bl_ref[...]
                     + _mm(oh_ref[...], vw_ref[...]))
    e2 = jnp.exp(h1 * t_ref[0, 0])
    h1_ref[...] = h1
    q_ref[0] = e2 * h1
    q_ref[1] = e2


def _sage1(parts, onehot, tnum, tden, wl, bl, vw, t):
    nb = _ACC_ROWS // _NBLK
    return pl.pallas_call(
        _sage1_body,
        grid=(nb,),
        in_specs=[
            pl.BlockSpec((2, _NBLK, 16), lambda i: (0, i, 0)),
            pl.BlockSpec((_NBLK, 16), lambda i: (i, 0)),
            pl.BlockSpec((16, 16), lambda i: (0, 0)),
            pl.BlockSpec((16, 16), lambda i: (0, 0)),
            pl.BlockSpec((16, 16), lambda i: (0, 0)),
            pl.BlockSpec((1, 16), lambda i: (0, 0)),
            pl.BlockSpec((16, 16), lambda i: (0, 0)),
            pl.BlockSpec((1, 1), lambda i: (0, 0), memory_space=pltpu.SMEM),
        ],
        out_specs=[
            pl.BlockSpec((_NBLK, 16), lambda i: (i, 0)),
            pl.BlockSpec((2, _NBLK, 16), lambda i: (0, i, 0)),
        ],
        out_shape=[
            jax.ShapeDtypeStruct((_ACC_ROWS, 16), jnp.float32),
            jax.ShapeDtypeStruct((2, _ACC_ROWS, 16), jnp.float32),
        ],
    )(parts, onehot, tnum, tden, wl, bl, vw, t)


def _sage2_body(parts_ref, h1_ref, wl_ref, bl_ref, wr_ref, h2_ref):
    aggr = parts_ref[0] / (parts_ref[1] + 1e-16)
    h2_ref[...] = jax.nn.relu(_mm(aggr, wl_ref[...]) + bl_ref[...]
                              + _mm(h1_ref[...], wr_ref[...]))


def _sage2(parts, h1, wl, bl, wr):
    nb = _ACC_ROWS // _NBLK
    return pl.pallas_call(
        _sage2_body,
        grid=(nb,),
        in_specs=[
            pl.BlockSpec((2, _NBLK, 16), lambda i: (0, i, 0)),
            pl.BlockSpec((_NBLK, 16), lambda i: (i, 0)),
            pl.BlockSpec((16, 16), lambda i: (0, 0)),
            pl.BlockSpec((1, 16), lambda i: (0, 0)),
            pl.BlockSpec((16, 16), lambda i: (0, 0)),
        ],
        out_specs=pl.BlockSpec((_NBLK, 16), lambda i: (i, 0)),
        out_shape=jax.ShapeDtypeStruct((_ACC_ROWS, 16), jnp.float32),
    )(parts, h1, wl, bl, wr)


def _gru_body(dense_ref, wis_ref, whs_ref, bi_ref, bh_ref, l_ref, out_ref):
    H = HIDDEN_C
    G = N_GRAPHS_C
    bi = bi_ref[...]
    bh = bh_ref[...]
    wis = [wis_ref[k] for k in range(H)]
    whs = [whs_ref[k] for k in range(H)]

    def step(tt, hT):
        xtT = dense_ref[tt]
        giT = jnp.zeros((3 * H, G), jnp.float32) + bi
        ghT = jnp.zeros((3 * H, G), jnp.float32) + bh
        for k in range(H):
            giT = giT + wis[k] * xtT[k:k + 1, :]
            ghT = ghT + whs[k] * hT[k:k + 1, :]
        r = jax.nn.sigmoid(giT[0:H] + ghT[0:H])
        z = jax.nn.sigmoid(giT[H:2 * H] + ghT[H:2 * H])
        n = jnp.tanh(giT[2 * H:3 * H] + r * ghT[2 * H:3 * H])
        return (1.0 - z) * n + z * hT

    h0 = jnp.zeros((H, G), jnp.float32)
    out_ref[...] = lax.fori_loop(0, l_ref[0, 0], step, h0)


def _gru(dense_tT, wi, wh, bi, bh, l_arr):
    return pl.pallas_call(
        _gru_body,
        in_specs=[
            pl.BlockSpec(dense_tT.shape, lambda: (0, 0, 0)),
            pl.BlockSpec((16, 48, 1), lambda: (0, 0, 0)),
            pl.BlockSpec((16, 48, 1), lambda: (0, 0, 0)),
            pl.BlockSpec((48, 1), lambda: (0, 0)),
            pl.BlockSpec((48, 1), lambda: (0, 0)),
            pl.BlockSpec((1, 1), lambda: (0, 0), memory_space=pltpu.SMEM),
        ],
        out_specs=pl.BlockSpec((HIDDEN_C, N_GRAPHS_C), lambda: (0, 0)),
        out_shape=jax.ShapeDtypeStruct((HIDDEN_C, N_GRAPHS_C), jnp.float32),
    )(dense_tT, wi[:, :, None], wh[:, :, None], bi[:, None], bh[:, None], l_arr)


_GBLK = 8


def _attn_body(dense_ref, counts_ref, wq, bq, wk, bk, wv, bv, wo, bo,
               plw, plb, s_, pwq, pbq, pwk, pbk, pwv, pbv, pwo, pbo, out_ref):
    i = pl.program_id(0)
    ME = MAX_ELEM_C
    sq = _mm(s_[...], pwq[...]) + pbq[...]
    for g in range(_GBLK):
        c = counts_ref[i * _GBLK + g]
        rowi = lax.broadcasted_iota(jnp.int32, (ME, 16), 0)
        x = jnp.where(rowi < c, dense_ref[g], 0.0)
        qp = _mm(x, wq[...]) + bq[...]
        kp = _mm(x, wk[...]) + bk[...]
        vp = _mm(x, wv[...]) + bv[...]
        scores = lax.dot_general(qp, kp, (((1,), (1,)), ((), ())),
                                 preferred_element_type=jnp.float32) * 0.25
        coli = lax.broadcasted_iota(jnp.int32, (ME, ME), 1)
        scores = jnp.where(coli < c, scores, -1e30)
        e = jnp.exp(scores)
        a = e / jnp.sum(e, axis=1, keepdims=True)
        out = qp + jnp.dot(a, vp, preferred_element_type=jnp.float32)
        z2 = out + jax.nn.relu(_mm(out, wo[...]) + bo[...])
        kv = jax.nn.relu(_mm(z2, plw[...]) + plb[...])
        kp2 = _mm(kv, pwk[...]) + pbk[...]
        vp2 = _mm(kv, pwv[...]) + pbv[...]
        s2 = _dot(sq, kp2, ((1,), (1,))) * 0.25
        coli2 = lax.broadcasted_iota(jnp.int32, (1, ME), 1)
        s2 = jnp.where(coli2 < c, s2, -1e30)
        m2 = jnp.max(s2, axis=1, keepdims=True)
        e2 = jnp.exp(s2 - m2)
        a2 = e2 / jnp.sum(e2, axis=1, keepdims=True)
        o2 = sq + _mm(a2, vp2)
        st = o2 + jax.nn.relu(_mm(o2, pwo[...]) + pbo[...])
        st = jnp.where(st != st, 0.0, jnp.clip(st, -3.402823e38, 3.402823e38))
        out_ref[pl.ds(g, 1), :] = st


def _attention(dense3, counts, p):
    nb = N_GRAPHS_C // _GBLK
    w16 = lambda: pl.BlockSpec((16, 16), lambda i: (0, 0))
    b16 = lambda: pl.BlockSpec((1, 16), lambda i: (0, 0))
    return pl.pallas_call(
        _attn_body,
        grid=(nb,),
        in_specs=[
            pl.BlockSpec((_GBLK, MAX_ELEM_C, 16), lambda i: (i, 0, 0)),
            pl.BlockSpec(memory_space=pltpu.SMEM),
            w16(), b16(), w16(), b16(), w16(), b16(), w16(), b16(),
            w16(), b16(), b16(), w16(), b16(), w16(), b16(), w16(), b16(),
            w16(), b16(),
        ],
        out_specs=pl.BlockSpec((_GBLK, 16), lambda i: (i, 0)),
        out_shape=jax.ShapeDtypeStruct((N_GRAPHS_C, 16), jnp.float32),
    )(dense3, counts,
      p['enc_Wq'], p['enc_bq'][None], p['enc_Wk'], p['enc_bk'][None],
      p['enc_Wv'], p['enc_bv'][None], p['enc_Wo'], p['enc_bo'][None],
      p['pma_lin_W'], p['pma_lin_b'][None],
      p['S'].reshape(1, 16), p['pma_Wq'], p['pma_bq'][None],
      p['pma_Wk'], p['pma_bk'][None], p['pma_Wv'], p['pma_bv'][None],
      p['pma_Wo'], p['pma_bo'][None])


def _mlp_final_body(df_ref, wmlp_ref, bmlp_ref, gru_ref, st_ref,
                    wf1_ref, wf2_ref, wf3_ref, bf_ref, out_ref):
    mlp = _mm(df_ref[...], wmlp_ref[...]) + bmlp_ref[...]
    out_ref[...] = (_mm(mlp, wf1_ref[...])
                    + _mm(gru_ref[...], wf2_ref[...])
                    + _mm(st_ref[...], wf3_ref[...])
                    + bf_ref[...])


def _mlp_final(dense_flat, wmlp, bmlp, gru, st, wfin, bfin):
    return pl.pallas_call(
        _mlp_final_body,
        out_shape=jax.ShapeDtypeStruct((N_GRAPHS_C, 48), jnp.float32),
    )(dense_flat, wmlp, bmlp[None], gru, st,
      wfin[0:16], wfin[16:32], wfin[32:48], bfin[None])


def kernel(params, x, edge_index, batch):
    p = params
    N = x.shape[0]
    G = N_GRAPHS_C
    ME = MAX_ELEM_C
    t = p['t']

    c0 = jnp.repeat(jnp.arange(3), 3)
    c1 = jnp.tile(jnp.arange(3), 3)
    V = jnp.concatenate([p['emb'][c0], c1[:, None].astype(jnp.float32)], axis=1)
    cls = x[:, 0] * 3 + x[:, 1]
    src, dst = edge_index[0], edge_index[1]
    src2d, dst2d = _pad_edges(src, dst)
    onehot = (cls[:, None] == jnp.arange(16)[None, :]).astype(jnp.float32)
    onehot = jnp.pad(onehot, ((0, _ACC_ROWS - N), (0, 0)))
    E1 = jnp.exp(V * t)
    tnum = jnp.zeros((16, 16), jnp.float32).at[:9, :4].set(E1 * V)
    tden = jnp.zeros((16, 16), jnp.float32).at[:9, :4].set(E1)
    wl1 = jnp.zeros((16, 16), jnp.float32).at[:4].set(p['Wl1'])
    vw = jnp.zeros((16, 16), jnp.float32).at[:9].set(V @ p['Wr1'])
    t11 = t.reshape(1, 1)

    counts, starts, l_arr = _graph_counts(batch)

    parts1 = _sc_edge_aggregate(src2d, dst2d, onehot[None], False)
    h1, q3 = _sage1(parts1, onehot, tnum, tden, wl1, p['bl1'][None], vw, t11)

    parts2 = _sc_edge_aggregate(src2d, dst2d, q3, True)
    h2 = _sage2(parts2, h1, p['Wl2'], p['bl2'][None], p['Wr2'])

    pidx = jnp.arange(ME)[None, :]
    gidx = starts[0][:, None] + pidx
    mask = pidx < counts[0][:, None]
    dense = jnp.where(mask[:, :, None],
                      h2[jnp.minimum(gidx, N - 1)], 0.0)

    gruT = _gru(jnp.transpose(dense, (1, 2, 0)), p['Wi'], p['Wh'], p['bi'],
                p['bh'], l_arr)
    st = _attention(dense, counts.reshape(G), p)
    return _mlp_final(dense.reshape(G, ME * HIDDEN_C), p['Wmlp'], p['bmlp'],
                      gruT.T, st, p['Wfin'], p['bfin'])
